# Initial kernel scaffold; baseline (speedup 1.0000x reference)
import jax
import jax.numpy as jnp
from jax.experimental import pallas as pl


def kernel(x, W, b):
    raise NotImplementedError("write your pallas kernel here")


# speedup vs baseline: 4.8052x; 4.8052x over previous
import functools

import jax
import jax.numpy as jnp
from jax import lax
from jax.experimental import pallas as pl
from jax.experimental.pallas import tpu as pltpu

N_TOK = 16384
D = 4096
E = 64
K = 8
M_BLK = 512


def _router_block(x_ref, wt_ref, b_ref, rw_ref, gates_ref):
    acc = jnp.dot(x_ref[...], wt_ref[...], preferred_element_type=jnp.float32)
    rw = acc + b_ref[...]
    rw_ref[...] = rw

    iota = lax.broadcasted_iota(jnp.int32, rw.shape, 1)
    cur = rw
    sel = jnp.zeros(rw.shape, jnp.bool_)
    m0 = jnp.max(rw, axis=1, keepdims=True)
    for _ in range(K):
        m = jnp.max(cur, axis=1, keepdims=True)
        cand = jnp.where(cur == m, iota, E)
        amin = jnp.min(cand, axis=1, keepdims=True)
        onehot = iota == amin
        sel = jnp.logical_or(sel, onehot)
        cur = jnp.where(onehot, -jnp.inf, cur)
    e = jnp.where(sel, jnp.exp(rw - m0), 0.0)
    s = jnp.sum(e, axis=1, keepdims=True)
    gates_ref[...] = e / s


@jax.jit
def kernel(x, W, b):
    wt = W.T
    b2 = b.reshape(1, E)
    grid = (N_TOK // M_BLK,)
    rw, gates = pl.pallas_call(
        _router_block,
        grid=grid,
        in_specs=[
            pl.BlockSpec((M_BLK, D), lambda i: (i, 0)),
            pl.BlockSpec((D, E), lambda i: (0, 0)),
            pl.BlockSpec((1, E), lambda i: (0, 0)),
        ],
        out_specs=[
            pl.BlockSpec((M_BLK, E), lambda i: (i, 0)),
            pl.BlockSpec((M_BLK, E), lambda i: (i, 0)),
        ],
        out_shape=[
            jax.ShapeDtypeStruct((N_TOK, E), jnp.float32),
            jax.ShapeDtypeStruct((N_TOK, E), jnp.float32),
        ],
        compiler_params=pltpu.CompilerParams(
            dimension_semantics=("arbitrary",),
        ),
    )(x, wt, b2)
    return (gates, rw)


# speedup vs baseline: 5.8373x; 1.2148x over previous
import functools

import jax
import jax.numpy as jnp
from jax import lax
from jax.experimental import pallas as pl
from jax.experimental.pallas import tpu as pltpu

N_TOK = 16384
D = 4096
E = 64
K = 8
M_BLK = 512


def _router_block(x_ref, wt_ref, b_ref, rw_ref, gates_ref):
    acc = jnp.dot(x_ref[...], wt_ref[...], preferred_element_type=jnp.float32)
    rw = acc + b_ref[...]
    rw_ref[...] = rw

    cur = rw
    t = jnp.max(cur, axis=1, keepdims=True)
    m0 = t
    for _ in range(K - 1):
        cur = jnp.where(cur == t, -jnp.inf, cur)
        t = jnp.max(cur, axis=1, keepdims=True)
    e = jnp.where(rw >= t, jnp.exp(rw - m0), 0.0)
    s = jnp.sum(e, axis=1, keepdims=True)
    gates_ref[...] = e / s


@jax.jit
def kernel(x, W, b):
    wt = W.T
    b2 = b.reshape(1, E)
    grid = (N_TOK // M_BLK,)
    rw, gates = pl.pallas_call(
        _router_block,
        grid=grid,
        in_specs=[
            pl.BlockSpec((M_BLK, D), lambda i: (i, 0)),
            pl.BlockSpec((D, E), lambda i: (0, 0)),
            pl.BlockSpec((1, E), lambda i: (0, 0)),
        ],
        out_specs=[
            pl.BlockSpec((M_BLK, E), lambda i: (i, 0)),
            pl.BlockSpec((M_BLK, E), lambda i: (i, 0)),
        ],
        out_shape=[
            jax.ShapeDtypeStruct((N_TOK, E), jnp.float32),
            jax.ShapeDtypeStruct((N_TOK, E), jnp.float32),
        ],
        compiler_params=pltpu.CompilerParams(
            dimension_semantics=("arbitrary",),
        ),
    )(x, wt, b2)
    return (gates, rw)


# speedup vs baseline: 6.2174x; 1.0651x over previous
import functools

import jax
import jax.numpy as jnp
from jax import lax
from jax.experimental import pallas as pl
from jax.experimental.pallas import tpu as pltpu

N_TOK = 16384
D = 4096
E = 64
K = 8
M_BLK = 1024


def _router_block(x_ref, wt_ref, b_ref, rw_ref, gates_ref):
    acc = jnp.dot(x_ref[...], wt_ref[...], preferred_element_type=jnp.float32)
    rw = acc + b_ref[...]
    rw_ref[...] = rw

    cur = rw
    t = jnp.max(cur, axis=1, keepdims=True)
    m0 = t
    for _ in range(K - 1):
        cur = jnp.where(cur == t, -jnp.inf, cur)
        t = jnp.max(cur, axis=1, keepdims=True)
    e = jnp.where(rw >= t, jnp.exp(rw - m0), 0.0)
    s = jnp.sum(e, axis=1, keepdims=True)
    gates_ref[...] = e / s


@jax.jit
def kernel(x, W, b):
    wt = W.T
    b2 = b.reshape(1, E)
    grid = (N_TOK // M_BLK,)
    rw, gates = pl.pallas_call(
        _router_block,
        grid=grid,
        in_specs=[
            pl.BlockSpec((M_BLK, D), lambda i: (i, 0)),
            pl.BlockSpec((D, E), lambda i: (0, 0)),
            pl.BlockSpec((1, E), lambda i: (0, 0)),
        ],
        out_specs=[
            pl.BlockSpec((M_BLK, E), lambda i: (i, 0)),
            pl.BlockSpec((M_BLK, E), lambda i: (i, 0)),
        ],
        out_shape=[
            jax.ShapeDtypeStruct((N_TOK, E), jnp.float32),
            jax.ShapeDtypeStruct((N_TOK, E), jnp.float32),
        ],
        compiler_params=pltpu.CompilerParams(
            dimension_semantics=("arbitrary",),
        ),
    )(x, wt, b2)
    return (gates, rw)
